# Initial kernel scaffold; baseline (speedup 1.0000x reference)
import jax
import jax.numpy as jnp
from jax.experimental import pallas as pl


def kernel(x, W, b_gcn, fc_W, fc_b):
    raise NotImplementedError("write your pallas kernel here")


# speedup vs baseline: 136.3967x; 136.3967x over previous
import jax
import jax.numpy as jnp
from jax.experimental import pallas as pl
from jax.experimental.pallas import tpu as pltpu

NODE = 256
REL = 4
QCOL = NODE * REL
BLK_J = 16
BLK_R = BLK_J * NODE
NSTEP = NODE // BLK_J


def _body(xm_ref, wt_ref, bcol_ref, fcb_ref, fcbias_ref, out_ref, outT_s):
    step = pl.program_id(0)

    @pl.when(step == 0)
    def _gcn():
        m = (xm_ref[...] != 0.0).astype(jnp.float32)
        qi = jax.lax.broadcasted_iota(jnp.int32, (QCOL, NODE), 0)
        ji = jax.lax.broadcasted_iota(jnp.int32, (QCOL, NODE), 1)
        sel = jnp.where((qi // REL) == ji, 1.0, 0.0)
        c = jnp.dot(m, sel, preferred_element_type=jnp.float32)
        deg = 1.0 + jnp.sum(c, axis=0, keepdims=True)
        dis = jax.lax.rsqrt(deg)
        wt = wt_ref[...]
        tT = jnp.dot(wt * dis, c, preferred_element_type=jnp.float32)
        outT_s[...] = dis * tT + (dis * dis) * wt + bcol_ref[...]

    ji2 = jax.lax.broadcasted_iota(jnp.int32, (NODE, BLK_J), 0)
    ti = jax.lax.broadcasted_iota(jnp.int32, (NODE, BLK_J), 1)
    sel_e = jnp.where(ji2 == step * BLK_J + ti, 1.0, 0.0)
    colblk = jnp.dot(outT_s[...], sel_e,
                     preferred_element_type=jnp.float32)

    fcb = fcb_ref[...]
    partial = jnp.zeros((1, NODE), jnp.float32)
    for jl in range(BLK_J):
        prod = colblk[:, jl:jl + 1] * fcb[jl * NODE:(jl + 1) * NODE, :]
        partial = partial + jnp.sum(prod, axis=0, keepdims=True)

    @pl.when(step == 0)
    def _init():
        out_ref[...] = partial + fcbias_ref[...]

    @pl.when(step > 0)
    def _acc():
        out_ref[...] = out_ref[...] + partial


def kernel(x, W, b_gcn, fc_W, fc_b):
    xm = x.reshape(NODE, QCOL)
    wt = W.T
    bcol = b_gcn.reshape(NODE, 1)
    fcbias = fc_b.reshape(1, NODE)
    return pl.pallas_call(
        _body,
        grid=(NSTEP,),
        in_specs=[
            pl.BlockSpec((NODE, QCOL), lambda s: (0, 0)),
            pl.BlockSpec((NODE, NODE), lambda s: (0, 0)),
            pl.BlockSpec((NODE, 1), lambda s: (0, 0)),
            pl.BlockSpec((BLK_R, NODE), lambda s: (s, 0)),
            pl.BlockSpec((1, NODE), lambda s: (0, 0)),
        ],
        out_specs=pl.BlockSpec((1, NODE), lambda s: (0, 0)),
        out_shape=jax.ShapeDtypeStruct((1, NODE), jnp.float32),
        scratch_shapes=[pltpu.VMEM((NODE, NODE), jnp.float32)],
    )(xm, wt, bcol, fc_W, fcbias)


# speedup vs baseline: 152.7522x; 1.1199x over previous
import jax
import jax.numpy as jnp
from jax.experimental import pallas as pl
from jax.experimental.pallas import tpu as pltpu

NODE = 256
REL = 4
QCOL = NODE * REL
BLK_J = 32
BLK_R = BLK_J * NODE
NSTEP = NODE // BLK_J


def _body(xm_ref, wt_ref, bcol_ref, fcb_ref, fcbias_ref, out_ref, outT_s):
    step = pl.program_id(0)

    @pl.when(step == 0)
    def _gcn():
        m = (xm_ref[...] != 0.0).astype(jnp.float32)
        qi = jax.lax.broadcasted_iota(jnp.int32, (QCOL, NODE), 0)
        ji = jax.lax.broadcasted_iota(jnp.int32, (QCOL, NODE), 1)
        sel = jnp.where((qi // REL) == ji, 1.0, 0.0)
        c = jnp.dot(m, sel, preferred_element_type=jnp.float32)
        deg = 1.0 + jnp.sum(c, axis=0, keepdims=True)
        dis = jax.lax.rsqrt(deg)
        wt = wt_ref[...]
        tT = jnp.dot(wt * dis, c, preferred_element_type=jnp.float32)
        outT_s[...] = dis * tT + (dis * dis) * wt + bcol_ref[...]

    ji2 = jax.lax.broadcasted_iota(jnp.int32, (NODE, BLK_J), 0)
    ti = jax.lax.broadcasted_iota(jnp.int32, (NODE, BLK_J), 1)
    sel_e = jnp.where(ji2 == step * BLK_J + ti, 1.0, 0.0)
    colblk = jnp.dot(outT_s[...], sel_e,
                     preferred_element_type=jnp.float32)

    fcb = fcb_ref[...]
    partial = jnp.zeros((1, NODE), jnp.float32)
    for jl in range(BLK_J):
        prod = colblk[:, jl:jl + 1] * fcb[jl * NODE:(jl + 1) * NODE, :]
        partial = partial + jnp.sum(prod, axis=0, keepdims=True)

    @pl.when(step == 0)
    def _init():
        out_ref[...] = partial + fcbias_ref[...]

    @pl.when(step > 0)
    def _acc():
        out_ref[...] = out_ref[...] + partial


def kernel(x, W, b_gcn, fc_W, fc_b):
    xm = x.reshape(NODE, QCOL)
    wt = W.T
    bcol = b_gcn.reshape(NODE, 1)
    fcbias = fc_b.reshape(1, NODE)
    return pl.pallas_call(
        _body,
        grid=(NSTEP,),
        in_specs=[
            pl.BlockSpec((NODE, QCOL), lambda s: (0, 0)),
            pl.BlockSpec((NODE, NODE), lambda s: (0, 0)),
            pl.BlockSpec((NODE, 1), lambda s: (0, 0)),
            pl.BlockSpec((BLK_R, NODE), lambda s: (s, 0)),
            pl.BlockSpec((1, NODE), lambda s: (0, 0)),
        ],
        out_specs=pl.BlockSpec((1, NODE), lambda s: (0, 0)),
        out_shape=jax.ShapeDtypeStruct((1, NODE), jnp.float32),
        scratch_shapes=[pltpu.VMEM((NODE, NODE), jnp.float32)],
    )(xm, wt, bcol, fc_W, fcbias)
